# Initial kernel scaffold; baseline (speedup 1.0000x reference)
import jax
import jax.numpy as jnp
from jax.experimental import pallas as pl


def kernel(doc_ids, context_ids, sample_ids, paragraph_matrix, word_matrix, outputs):
    raise NotImplementedError("write your pallas kernel here")


# speedup vs baseline: 1.2278x; 1.2278x over previous
import functools

import jax
import jax.numpy as jnp
from jax import lax
from jax.experimental import pallas as pl
from jax.experimental.pallas import tpu as pltpu
from jax.experimental.pallas import tpu_sc as plsc

B = 16384
CTX = 8
NSAMP = 10
D = 64
NT = D // 16
NC = 2
NS = 16
NW = NC * NS
BW = B // NW
CHUNK = 64
NCHUNK = BW // CHUNK


def _body(doc_hbm, ctx_hbm, smp_hbm, par_hbm, word_hbm, outT_hbm, res_hbm,
          doc_idx, ctx_idx, smp_idx, doc_rows, ctx_rows, smp_rows, res_v, sem):
    wid = lax.axis_index("s") * NC + lax.axis_index("c")

    def chunk_body(k, carry):
        base = wid * BW + k * CHUNK
        pltpu.sync_copy(doc_hbm.at[pl.ds(base, CHUNK)], doc_idx)
        pltpu.sync_copy(ctx_hbm.at[pl.ds(base * CTX, CHUNK * CTX)], ctx_idx)
        pltpu.sync_copy(smp_hbm.at[pl.ds(base * NSAMP, CHUNK * NSAMP)], smp_idx)
        copies = [pltpu.async_copy(par_hbm.at[doc_idx], doc_rows, sem)]
        for j in range(CHUNK * CTX // 128):
            copies.append(pltpu.async_copy(
                word_hbm.at[ctx_idx.at[pl.ds(j * 128, 128)]],
                ctx_rows.at[pl.ds(j * 128, 128)], sem))
        for j in range(CHUNK * NSAMP // 128):
            copies.append(pltpu.async_copy(
                outT_hbm.at[smp_idx.at[pl.ds(j * 128, 128)]],
                smp_rows.at[pl.ds(j * 128, 128)], sem))
        for c in copies:
            c.wait()

        lanes = lax.iota(jnp.int32, 16)

        def elem_body(e, carry2):
            inp = []
            for t in range(NT):
                a = ctx_rows[e * CTX, pl.ds(t * 16, 16)]
                for c in range(1, CTX):
                    a = a + ctx_rows[e * CTX + c, pl.ds(t * 16, 16)]
                inp.append(doc_rows[e, pl.ds(t * 16, 16)] + a * (1.0 / CTX))
            acc = jnp.zeros((16,), jnp.float32)
            for s in range(NSAMP):
                r = inp[0] * smp_rows[e * NSAMP + s, pl.ds(0, 16)]
                for t in range(1, NT):
                    r = r + inp[t] * smp_rows[e * NSAMP + s, pl.ds(t * 16, 16)]
                acc = jnp.where(lanes == s, jnp.sum(r), acc)
            plsc.store_scatter(res_v, [e * NSAMP + lanes], acc,
                               mask=lanes < NSAMP)
            return carry2

        lax.fori_loop(0, CHUNK, elem_body, 0)
        pltpu.sync_copy(res_v, res_hbm.at[pl.ds(base * NSAMP, CHUNK * NSAMP)])
        return carry

    lax.fori_loop(0, NCHUNK, chunk_body, 0)


_sc_call = functools.partial(
    pl.kernel,
    out_type=jax.ShapeDtypeStruct((B * NSAMP,), jnp.float32),
    mesh=plsc.VectorSubcoreMesh(core_axis_name="c", subcore_axis_name="s"),
    compiler_params=pltpu.CompilerParams(needs_layout_passes=False,
                                         use_tc_tiling_on_sc=False),
    scratch_types=[
        pltpu.VMEM((CHUNK,), jnp.int32),
        pltpu.VMEM((CHUNK * CTX,), jnp.int32),
        pltpu.VMEM((CHUNK * NSAMP,), jnp.int32),
        pltpu.VMEM((CHUNK, D), jnp.float32),
        pltpu.VMEM((CHUNK * CTX, D), jnp.float32),
        pltpu.VMEM((CHUNK * NSAMP, D), jnp.float32),
        pltpu.VMEM((CHUNK * NSAMP,), jnp.float32),
        pltpu.SemaphoreType.DMA,
    ],
)(_body)


def kernel(doc_ids, context_ids, sample_ids, paragraph_matrix, word_matrix,
           outputs):
    doc32 = doc_ids.astype(jnp.int32)
    ctx_flat = context_ids.reshape(B * CTX)
    smp_flat = sample_ids.reshape(B * NSAMP)
    outT = outputs.T
    res = _sc_call(doc32, ctx_flat, smp_flat, paragraph_matrix, word_matrix,
                   outT)
    return res.reshape(B, NSAMP)


# speedup vs baseline: 1.2487x; 1.0170x over previous
import functools

import jax
import jax.numpy as jnp
from jax import lax
from jax.experimental import pallas as pl
from jax.experimental.pallas import tpu as pltpu
from jax.experimental.pallas import tpu_sc as plsc

B = 16384
CTX = 8
NSAMP = 10
D = 64
NT = D // 16
NC = 2
NS = 16
NW = NC * NS
BW = B // NW
CHUNK = 64
NCHUNK = BW // CHUNK


def _body(doc_hbm, ctx_hbm, smp_hbm, par_hbm, word_hbm, outT_hbm, res_hbm,
          doc_idx, ctx_idx, smp_idx, doc_rows, ctx_rows, smp_rows, res_v, sem):
    wid = lax.axis_index("s") * NC + lax.axis_index("c")
    lanes = lax.iota(jnp.int32, 16)

    def chunk_body(k, carry):
        base = wid * BW + k * CHUNK
        pltpu.sync_copy(doc_hbm.at[pl.ds(base, CHUNK)], doc_idx)
        pltpu.sync_copy(ctx_hbm.at[:, pl.ds(base, CHUNK)], ctx_idx)
        pltpu.sync_copy(smp_hbm.at[:, pl.ds(base, CHUNK)], smp_idx)
        copies = [pltpu.async_copy(par_hbm.at[doc_idx], doc_rows, sem)]
        for c in range(CTX):
            copies.append(pltpu.async_copy(
                word_hbm.at[ctx_idx.at[c]],
                ctx_rows.at[pl.ds(c * CHUNK, CHUNK)], sem))
        for s in range(NSAMP):
            copies.append(pltpu.async_copy(
                outT_hbm.at[smp_idx.at[s]],
                smp_rows.at[pl.ds(s * CHUNK, CHUNK)], sem))
        for cp in copies:
            cp.wait()

        def elem_body(e, carry2):
            inp = []
            for t in range(NT):
                a = ctx_rows[e, pl.ds(t * 16, 16)]
                for c in range(1, CTX):
                    a = a + ctx_rows[c * CHUNK + e, pl.ds(t * 16, 16)]
                inp.append(doc_rows[e, pl.ds(t * 16, 16)] + a * (1.0 / CTX))
            acc = jnp.zeros((16,), jnp.float32)
            for s in range(NSAMP):
                r = inp[0] * smp_rows[s * CHUNK + e, pl.ds(0, 16)]
                for t in range(1, NT):
                    r = r + inp[t] * smp_rows[s * CHUNK + e, pl.ds(t * 16, 16)]
                acc = jnp.where(lanes == s, jnp.sum(r), acc)
            plsc.store_scatter(res_v, [jnp.full((16,), e, jnp.int32), lanes],
                               acc, mask=lanes < NSAMP)
            return carry2

        lax.fori_loop(0, CHUNK, elem_body, 0)
        pltpu.sync_copy(res_v, res_hbm.at[pl.ds(base, CHUNK)])
        return carry

    lax.fori_loop(0, NCHUNK, chunk_body, 0)


_sc_call = functools.partial(
    pl.kernel,
    out_type=jax.ShapeDtypeStruct((B, NSAMP), jnp.float32),
    mesh=plsc.VectorSubcoreMesh(core_axis_name="c", subcore_axis_name="s"),
    compiler_params=pltpu.CompilerParams(needs_layout_passes=False,
                                         use_tc_tiling_on_sc=False),
    scratch_types=[
        pltpu.VMEM((CHUNK,), jnp.int32),
        pltpu.VMEM((CTX, CHUNK), jnp.int32),
        pltpu.VMEM((NSAMP, CHUNK), jnp.int32),
        pltpu.VMEM((CHUNK, D), jnp.float32),
        pltpu.VMEM((CTX * CHUNK, D), jnp.float32),
        pltpu.VMEM((NSAMP * CHUNK, D), jnp.float32),
        pltpu.VMEM((CHUNK, NSAMP), jnp.float32),
        pltpu.SemaphoreType.DMA,
    ],
)(_body)


def kernel(doc_ids, context_ids, sample_ids, paragraph_matrix, word_matrix,
           outputs):
    doc32 = doc_ids.astype(jnp.int32)
    return _sc_call(doc32, context_ids.T, sample_ids.T, paragraph_matrix,
                    word_matrix, outputs.T)
